# baseline (speedup 1.0000x reference)
import functools

import jax
import jax.numpy as jnp
from jax.experimental import pallas as pl
from jax.experimental.pallas import tpu as pltpu
from jax.experimental.pallas import tpu_sc as plsc

DEPTH = 11
N_NODES = 2 ** DEPTH - 1
N_LEAVES = 2 ** DEPTH
D = 1024
H = 16
OUT = 1024
BATCH = 8192
BLK = 1024
RGRP = 64
LPB = 16
NLB = N_LEAVES // LPB
CH = 64
OFF_PAD = N_LEAVES + 128
XPAD = BATCH + NLB * CH + CH
NCMAX = BATCH // CH

SC_NC = 2
SC_NS = 16
SC_NW = SC_NC * SC_NS
TPW = BATCH // SC_NW
SC_CH = 32
SC_NCH = TPW // SC_CH


def _routing_body(x_ref, nw_ref, nb_ref, leaves_ref, rank_ref, off_ref, cnt_ref):
    b = pl.program_id(0)

    @pl.when(b == 0)
    def _():
        cnt_ref[...] = jnp.zeros_like(cnt_ref)

    x = x_ref[...]
    scores = jax.lax.dot_general(
        x, nw_ref[...], (((1,), (1,)), ((), ())),
        precision=jax.lax.Precision.HIGHEST,
        preferred_element_type=jnp.float32)
    scores = scores + nb_ref[...]
    col = jax.lax.broadcasted_iota(jnp.int32, (BLK, N_LEAVES), 1)
    cur = jnp.zeros((BLK, 1), jnp.int32)
    for i in range(DEPTH):
        onehot = (col == cur).astype(jnp.float32)
        s = jnp.sum(scores * onehot, axis=1, keepdims=True)
        choice = (s >= 0).astype(jnp.int32)
        platform = 2 ** i - 1
        next_platform = 2 ** (i + 1) - 1
        cur = (cur - platform) * 2 + choice + next_platform
    leaf = cur - N_NODES
    leaves_ref[...] = leaf

    cnt = cnt_ref[...]
    colg = jax.lax.broadcasted_iota(jnp.int32, (RGRP, N_LEAVES), 1)
    rr = jax.lax.broadcasted_iota(jnp.int32, (RGRP, RGRP), 0)
    rc = jax.lax.broadcasted_iota(jnp.int32, (RGRP, RGRP), 1)
    ranks = []
    for g in range(BLK // RGRP):
        lv = jax.lax.slice(leaf, (g * RGRP, 0), ((g + 1) * RGRP, 1))
        m = (colg == lv).astype(jnp.float32)
        prior = jnp.sum(m * cnt, axis=1, keepdims=True)
        eq = (lv == jnp.reshape(lv, (1, RGRP))) & (rr > rc)
        intra = jnp.sum(eq.astype(jnp.float32), axis=1, keepdims=True)
        ranks.append(prior + intra)
        cnt = cnt + jnp.sum(m, axis=0, keepdims=True)
    cnt_ref[...] = cnt
    rank_ref[...] = jnp.concatenate(ranks, axis=0).astype(jnp.int32)

    @pl.when(b == BATCH // BLK - 1)
    def _():
        c = cnt_ref[...]
        lane = jax.lax.broadcasted_iota(jnp.int32, (1, N_LEAVES), 1)
        lmod = lane % LPB

        def shift_from_left(v, k):
            return jnp.concatenate(
                [jnp.zeros((1, k), jnp.float32), jax.lax.slice(v, (0, 0), (1, N_LEAVES - k))],
                axis=1)

        def shift_from_right(v, k):
            return jnp.concatenate(
                [jax.lax.slice(v, (0, k), (1, N_LEAVES)), jnp.zeros((1, k), jnp.float32)],
                axis=1)

        s = c
        k = 1
        while k < LPB:
            s = s + jnp.where(lmod >= k, shift_from_left(s, k), 0.0)
            k *= 2
        excl_within = s - c
        t = s
        k = 1
        while k < LPB:
            t = jnp.maximum(t, jnp.where(lmod + k < LPB, shift_from_right(t, k), 0.0))
            k *= 2
        tpad = jnp.ceil(t / float(CH)) * float(CH)
        u = jnp.where(lmod == LPB - 1, tpad, 0.0)
        acc = u
        k = 1
        while k < N_LEAVES:
            acc = acc + shift_from_left(acc, k)
            k *= 2
        excl_u = acc - u
        off = excl_u + excl_within
        total = jnp.sum(u)
        pad = jnp.full((1, OFF_PAD - N_LEAVES), 1.0, jnp.float32) * total
        off_ref[...] = jnp.concatenate([off, pad], axis=1).astype(jnp.int32)


def _route(x, node_weights, node_biases):
    nw_pad = jnp.concatenate(
        [node_weights, jnp.zeros((1, D), node_weights.dtype)], axis=0)
    nb_pad = jnp.concatenate(
        [node_biases[:, 0], jnp.zeros((1,), node_biases.dtype)]).reshape(1, N_LEAVES)
    leaves, rank, off = pl.pallas_call(
        _routing_body,
        grid=(BATCH // BLK,),
        in_specs=[
            pl.BlockSpec((BLK, D), lambda b: (b, 0)),
            pl.BlockSpec((N_LEAVES, D), lambda b: (0, 0)),
            pl.BlockSpec((1, N_LEAVES), lambda b: (0, 0)),
        ],
        out_specs=[
            pl.BlockSpec((BLK, 1), lambda b: (b, 0)),
            pl.BlockSpec((BLK, 1), lambda b: (b, 0)),
            pl.BlockSpec((1, OFF_PAD), lambda b: (0, 0)),
        ],
        out_shape=[
            jax.ShapeDtypeStruct((BATCH, 1), jnp.int32),
            jax.ShapeDtypeStruct((BATCH, 1), jnp.int32),
            jax.ShapeDtypeStruct((1, OFF_PAD), jnp.int32),
        ],
        scratch_shapes=[pltpu.VMEM((1, N_LEAVES), jnp.float32)],
    )(x, nw_pad, nb_pad)
    return leaves, rank, off


def _dispatch(x, leaves, rank, offsets):
    mesh = plsc.VectorSubcoreMesh(core_axis_name="c", subcore_axis_name="s")

    @functools.partial(
        pl.kernel,
        mesh=mesh,
        out_type=[
            jax.ShapeDtypeStruct((XPAD, D), jnp.float32),
            jax.ShapeDtypeStruct((BATCH,), jnp.int32),
        ],
        scratch_types=[
            pltpu.VMEM((N_LEAVES,), jnp.int32),
            pltpu.VMEM((SC_CH, D), jnp.float32),
            pltpu.VMEM((SC_NCH, SC_CH), jnp.int32),
            pltpu.VMEM((SC_CH,), jnp.int32),
            pltpu.VMEM((SC_CH,), jnp.int32),
            pltpu.SemaphoreType.DMA,
        ],
        compiler_params=pltpu.CompilerParams(needs_layout_passes=False),
    )
    def body(x_hbm, lv_hbm, rk_hbm, off_hbm, xs_hbm, pos_hbm,
             off_v, rows_v, pos_v, lv_v, rk_v, sem):
        wid = jax.lax.axis_index("s") * SC_NC + jax.lax.axis_index("c")
        pltpu.sync_copy(off_hbm, off_v)
        for i in range(SC_NCH):
            base = wid * TPW + i * SC_CH
            pltpu.sync_copy(lv_hbm.at[pl.ds(base, SC_CH)], lv_v)
            pltpu.sync_copy(rk_hbm.at[pl.ds(base, SC_CH)], rk_v)
            for k in range(0, SC_CH, 16):
                lvv = lv_v[pl.ds(k, 16)]
                offv = plsc.load_gather(off_v, [lvv])
                pos_v[i, pl.ds(k, 16)] = offv + rk_v[pl.ds(k, 16)]
            pltpu.sync_copy(x_hbm.at[pl.ds(base, SC_CH)], rows_v)
            pltpu.async_copy(rows_v, xs_hbm.at[pos_v.at[i]], sem).wait()
            pltpu.sync_copy(pos_v.at[i], pos_hbm.at[pl.ds(base, SC_CH)])

    return body(x, leaves, rank, offsets)


def _expert_body(off_ref, w1_ref, w2_ref, b1_ref, xs_ref, os_ref):
    L = pl.program_id(0)
    c = pl.program_id(1)
    s0 = off_ref[LPB * L]
    n = (off_ref[LPB * L + LPB] - s0) // CH

    @pl.when(c < n)
    def _():
        base = s0 + c * CH
        y = jax.lax.dot_general(
            xs_ref[...], w1_ref[...], (((1,), (0,)), ((), ())),
            preferred_element_type=jnp.float32)
        y = y + b1_ref[0]
        a = jax.nn.gelu(y)
        g = base + jax.lax.broadcasted_iota(jnp.int32, (CH, 1), 0)
        masks = []
        for j in range(LPB):
            mj = (g >= off_ref[LPB * L + j]) & (g < off_ref[LPB * L + j + 1])
            masks.append(jnp.broadcast_to(mj, (CH, H)).astype(jnp.float32))
        mask = jnp.concatenate(masks, axis=1)
        am = a * mask
        os_ref[...] = jax.lax.dot_general(
            am, w2_ref[...], (((1,), (0,)), ((), ())),
            preferred_element_type=jnp.float32)


def _row_block_map(L, c, off):
    s0 = off[LPB * L]
    n = (off[LPB * L + LPB] - s0) // CH
    return s0 // CH + jnp.minimum(c, jnp.maximum(n - 1, 0))


def _experts(off_flat, w1t, w2r, b1r, xs):
    grid_spec = pltpu.PrefetchScalarGridSpec(
        num_scalar_prefetch=1,
        grid=(NLB, NCMAX),
        in_specs=[
            pl.BlockSpec((D, LPB * H), lambda L, c, off: (0, L)),
            pl.BlockSpec((LPB * H, OUT), lambda L, c, off: (L, 0)),
            pl.BlockSpec((1, 1, LPB * H), lambda L, c, off: (L, 0, 0)),
            pl.BlockSpec((CH, D), lambda L, c, off: (_row_block_map(L, c, off), 0)),
        ],
        out_specs=pl.BlockSpec((CH, OUT), lambda L, c, off: (_row_block_map(L, c, off), 0)),
    )
    return pl.pallas_call(
        _expert_body,
        grid_spec=grid_spec,
        out_shape=jax.ShapeDtypeStruct((XPAD, OUT), jnp.float32),
    )(off_flat, w1t, w2r, b1r, xs)


def _undispatch(os_full, pos):
    mesh = plsc.VectorSubcoreMesh(core_axis_name="c", subcore_axis_name="s")

    @functools.partial(
        pl.kernel,
        mesh=mesh,
        out_type=jax.ShapeDtypeStruct((BATCH, OUT), jnp.float32),
        scratch_types=[
            pltpu.VMEM((SC_CH, OUT), jnp.float32),
            pltpu.VMEM((SC_CH,), jnp.int32),
            pltpu.SemaphoreType.DMA,
        ],
    )
    def body(os_hbm, pos_hbm, out_hbm, rows_v, pos_v, sem):
        wid = jax.lax.axis_index("s") * SC_NC + jax.lax.axis_index("c")
        for i in range(SC_NCH):
            base = wid * TPW + i * SC_CH
            pltpu.sync_copy(pos_hbm.at[pl.ds(base, SC_CH)], pos_v)
            pltpu.async_copy(os_hbm.at[pos_v], rows_v, sem).wait()
            pltpu.sync_copy(rows_v, out_hbm.at[pl.ds(base, SC_CH)])

    return body(os_full, pos)


def kernel(x, node_weights, node_biases, w1s, b1s, w2s):
    leaves2d, rank2d, off2d = _route(x, node_weights, node_biases)
    leaves = leaves2d[:, 0]
    rank = rank2d[:, 0]
    off_flat = off2d[0]
    xs, pos = _dispatch(x, leaves, rank, off_flat[:N_LEAVES])
    w1t = w1s.transpose(1, 0, 2).reshape(D, N_LEAVES * H)
    w2r = w2s.reshape(N_LEAVES * H, OUT)
    b1r = b1s.reshape(NLB, 1, LPB * H)
    os_full = _experts(off_flat, w1t, w2r, b1r, xs)
    return _undispatch(os_full, pos)


# speedup vs baseline: 3.8118x; 3.8118x over previous
import functools

import jax
import jax.numpy as jnp
from jax.experimental import pallas as pl
from jax.experimental.pallas import tpu as pltpu
from jax.experimental.pallas import tpu_sc as plsc

DEPTH = 11
N_NODES = 2 ** DEPTH - 1
N_LEAVES = 2 ** DEPTH
D = 1024
H = 16
OUT = 1024
BATCH = 8192
BLK = 1024
RGRP = 64
LPB = 16
NLB = N_LEAVES // LPB
CH = 64
OFF_PAD = N_LEAVES + 128
XPAD = BATCH + NLB * CH + CH
NCHUNKS = 256

SC_NC = 2
SC_NS = 16
SC_NW = SC_NC * SC_NS
TPW = BATCH // SC_NW
SC_CH = 32
SC_NCH = TPW // SC_CH


def _routing_body(x_ref, nw_ref, nb_ref, leaves_ref, rank_ref, off_ref, cbm_ref, cnt_ref):
    b = pl.program_id(0)

    @pl.when(b == 0)
    def _():
        cnt_ref[...] = jnp.zeros_like(cnt_ref)

    x = x_ref[...]
    scores = jax.lax.dot_general(
        x, nw_ref[...], (((1,), (1,)), ((), ())),
        precision=jax.lax.Precision.HIGHEST,
        preferred_element_type=jnp.float32)
    scores = scores + nb_ref[...]
    col = jax.lax.broadcasted_iota(jnp.int32, (BLK, N_LEAVES), 1)
    cur = jnp.zeros((BLK, 1), jnp.int32)
    for i in range(DEPTH):
        onehot = (col == cur).astype(jnp.float32)
        s = jnp.sum(scores * onehot, axis=1, keepdims=True)
        choice = (s >= 0).astype(jnp.int32)
        platform = 2 ** i - 1
        next_platform = 2 ** (i + 1) - 1
        cur = (cur - platform) * 2 + choice + next_platform
    leaf = cur - N_NODES
    leaves_ref[...] = leaf

    cnt = cnt_ref[...]
    colg = jax.lax.broadcasted_iota(jnp.int32, (RGRP, N_LEAVES), 1)
    rr = jax.lax.broadcasted_iota(jnp.int32, (RGRP, RGRP), 0)
    rc = jax.lax.broadcasted_iota(jnp.int32, (RGRP, RGRP), 1)
    ranks = []
    for g in range(BLK // RGRP):
        lv = jax.lax.slice(leaf, (g * RGRP, 0), ((g + 1) * RGRP, 1))
        m = (colg == lv).astype(jnp.float32)
        prior = jnp.sum(m * cnt, axis=1, keepdims=True)
        eq = (lv == jnp.reshape(lv, (1, RGRP))) & (rr > rc)
        intra = jnp.sum(eq.astype(jnp.float32), axis=1, keepdims=True)
        ranks.append(prior + intra)
        cnt = cnt + jnp.sum(m, axis=0, keepdims=True)
    cnt_ref[...] = cnt
    rank_ref[...] = jnp.concatenate(ranks, axis=0).astype(jnp.int32)

    @pl.when(b == BATCH // BLK - 1)
    def _():
        c = cnt_ref[...]
        lane = jax.lax.broadcasted_iota(jnp.int32, (1, N_LEAVES), 1)
        lmod = lane % LPB

        def shift_from_left(v, k):
            return jnp.concatenate(
                [jnp.zeros((1, k), jnp.float32), jax.lax.slice(v, (0, 0), (1, N_LEAVES - k))],
                axis=1)

        def shift_from_right(v, k):
            return jnp.concatenate(
                [jax.lax.slice(v, (0, k), (1, N_LEAVES)), jnp.zeros((1, k), jnp.float32)],
                axis=1)

        s = c
        k = 1
        while k < LPB:
            s = s + jnp.where(lmod >= k, shift_from_left(s, k), 0.0)
            k *= 2
        excl_within = s - c
        t = s
        k = 1
        while k < LPB:
            t = jnp.maximum(t, jnp.where(lmod + k < LPB, shift_from_right(t, k), 0.0))
            k *= 2
        tpad = jnp.ceil(t / float(CH)) * float(CH)
        u = jnp.where(lmod == LPB - 1, tpad, 0.0)
        acc = u
        k = 1
        while k < N_LEAVES:
            acc = acc + shift_from_left(acc, k)
            k *= 2
        excl_u = acc - u
        off = excl_u + excl_within
        total = jnp.sum(u)
        pad = jnp.full((1, OFF_PAD - N_LEAVES), 1.0, jnp.float32) * total
        off_ref[...] = jnp.concatenate([off, pad], axis=1).astype(jnp.int32)
        kvec = jax.lax.broadcasted_iota(jnp.int32, (NCHUNKS, 1), 0).astype(jnp.float32) * float(CH)
        is_start = (lmod == 0)
        le = ((excl_u <= kvec) & is_start).astype(jnp.float32)
        cbm_ref[...] = (jnp.sum(le, axis=1, keepdims=True) - 1.0).astype(jnp.int32)


def _route(x, node_weights, node_biases):
    nw_pad = jnp.concatenate(
        [node_weights, jnp.zeros((1, D), node_weights.dtype)], axis=0)
    nb_pad = jnp.concatenate(
        [node_biases[:, 0], jnp.zeros((1,), node_biases.dtype)]).reshape(1, N_LEAVES)
    leaves, rank, off, cbm = pl.pallas_call(
        _routing_body,
        grid=(BATCH // BLK,),
        in_specs=[
            pl.BlockSpec((BLK, D), lambda b: (b, 0)),
            pl.BlockSpec((N_LEAVES, D), lambda b: (0, 0)),
            pl.BlockSpec((1, N_LEAVES), lambda b: (0, 0)),
        ],
        out_specs=[
            pl.BlockSpec((BLK, 1), lambda b: (b, 0)),
            pl.BlockSpec((BLK, 1), lambda b: (b, 0)),
            pl.BlockSpec((1, OFF_PAD), lambda b: (0, 0)),
            pl.BlockSpec((NCHUNKS, 1), lambda b: (0, 0)),
        ],
        out_shape=[
            jax.ShapeDtypeStruct((BATCH, 1), jnp.int32),
            jax.ShapeDtypeStruct((BATCH, 1), jnp.int32),
            jax.ShapeDtypeStruct((1, OFF_PAD), jnp.int32),
            jax.ShapeDtypeStruct((NCHUNKS, 1), jnp.int32),
        ],
        scratch_shapes=[pltpu.VMEM((1, N_LEAVES), jnp.float32)],
    )(x, nw_pad, nb_pad)
    return leaves, rank, off, cbm


def _dispatch(x, leaves, rank, offsets):
    mesh = plsc.VectorSubcoreMesh(core_axis_name="c", subcore_axis_name="s")

    @functools.partial(
        pl.kernel,
        mesh=mesh,
        out_type=[
            jax.ShapeDtypeStruct((XPAD, D), jnp.float32),
            jax.ShapeDtypeStruct((BATCH,), jnp.int32),
        ],
        scratch_types=[
            pltpu.VMEM((N_LEAVES,), jnp.int32),
            pltpu.VMEM((SC_CH, D), jnp.float32),
            pltpu.VMEM((SC_NCH, SC_CH), jnp.int32),
            pltpu.VMEM((SC_CH,), jnp.int32),
            pltpu.VMEM((SC_CH,), jnp.int32),
            pltpu.SemaphoreType.DMA,
        ],
        compiler_params=pltpu.CompilerParams(needs_layout_passes=False),
    )
    def body(x_hbm, lv_hbm, rk_hbm, off_hbm, xs_hbm, pos_hbm,
             off_v, rows_v, pos_v, lv_v, rk_v, sem):
        wid = jax.lax.axis_index("s") * SC_NC + jax.lax.axis_index("c")
        pltpu.sync_copy(off_hbm, off_v)
        for i in range(SC_NCH):
            base = wid * TPW + i * SC_CH
            pltpu.sync_copy(lv_hbm.at[pl.ds(base, SC_CH)], lv_v)
            pltpu.sync_copy(rk_hbm.at[pl.ds(base, SC_CH)], rk_v)
            for k in range(0, SC_CH, 16):
                lvv = lv_v[pl.ds(k, 16)]
                offv = plsc.load_gather(off_v, [lvv])
                pos_v[i, pl.ds(k, 16)] = offv + rk_v[pl.ds(k, 16)]
            pltpu.sync_copy(x_hbm.at[pl.ds(base, SC_CH)], rows_v)
            pltpu.async_copy(rows_v, xs_hbm.at[pos_v.at[i]], sem).wait()
            pltpu.sync_copy(pos_v.at[i], pos_hbm.at[pl.ds(base, SC_CH)])

    return body(x, leaves, rank, offsets)


def _expert_body(off_ref, cbm_ref, w1_ref, w2_ref, b1_ref, xs_ref, os_ref):
    k = pl.program_id(0)
    L = cbm_ref[k]
    base = k * CH
    y = jax.lax.dot_general(
        xs_ref[...], w1_ref[...], (((1,), (0,)), ((), ())),
        preferred_element_type=jnp.float32)
    y = y + b1_ref[0]
    a = jax.nn.gelu(y)
    g = base + jax.lax.broadcasted_iota(jnp.int32, (CH, 1), 0)
    masks = []
    for j in range(LPB):
        mj = (g >= off_ref[LPB * L + j]) & (g < off_ref[LPB * L + j + 1])
        masks.append(jnp.broadcast_to(mj, (CH, H)).astype(jnp.float32))
    mask = jnp.concatenate(masks, axis=1)
    am = a * mask
    os_ref[...] = jax.lax.dot_general(
        am, w2_ref[...], (((1,), (0,)), ((), ())),
        preferred_element_type=jnp.float32)


def _experts(off_flat, cbm_flat, w1t, w2r, b1r, xs):
    grid_spec = pltpu.PrefetchScalarGridSpec(
        num_scalar_prefetch=2,
        grid=(NCHUNKS,),
        in_specs=[
            pl.BlockSpec((D, LPB * H), lambda k, off, cbm: (0, cbm[k])),
            pl.BlockSpec((LPB * H, OUT), lambda k, off, cbm: (cbm[k], 0)),
            pl.BlockSpec((1, 1, LPB * H), lambda k, off, cbm: (cbm[k], 0, 0)),
            pl.BlockSpec((CH, D), lambda k, off, cbm: (k, 0)),
        ],
        out_specs=pl.BlockSpec((CH, OUT), lambda k, off, cbm: (k, 0)),
    )
    return pl.pallas_call(
        _expert_body,
        grid_spec=grid_spec,
        out_shape=jax.ShapeDtypeStruct((XPAD, OUT), jnp.float32),
    )(off_flat, cbm_flat, w1t, w2r, b1r, xs)


def _undispatch(os_full, pos):
    mesh = plsc.VectorSubcoreMesh(core_axis_name="c", subcore_axis_name="s")

    @functools.partial(
        pl.kernel,
        mesh=mesh,
        out_type=jax.ShapeDtypeStruct((BATCH, OUT), jnp.float32),
        scratch_types=[
            pltpu.VMEM((SC_CH, OUT), jnp.float32),
            pltpu.VMEM((SC_CH,), jnp.int32),
            pltpu.SemaphoreType.DMA,
        ],
    )
    def body(os_hbm, pos_hbm, out_hbm, rows_v, pos_v, sem):
        wid = jax.lax.axis_index("s") * SC_NC + jax.lax.axis_index("c")
        for i in range(SC_NCH):
            base = wid * TPW + i * SC_CH
            pltpu.sync_copy(pos_hbm.at[pl.ds(base, SC_CH)], pos_v)
            pltpu.async_copy(os_hbm.at[pos_v], rows_v, sem).wait()
            pltpu.sync_copy(rows_v, out_hbm.at[pl.ds(base, SC_CH)])

    return body(os_full, pos)


def kernel(x, node_weights, node_biases, w1s, b1s, w2s):
    leaves2d, rank2d, off2d, cbm2d = _route(x, node_weights, node_biases)
    leaves = leaves2d[:, 0]
    rank = rank2d[:, 0]
    off_flat = off2d[0]
    xs, pos = _dispatch(x, leaves, rank, off_flat[:N_LEAVES])
    w1t = w1s.transpose(1, 0, 2).reshape(D, N_LEAVES * H)
    w2r = w2s.reshape(N_LEAVES * H, OUT)
    b1r = b1s.reshape(NLB, 1, LPB * H)
    os_full = _experts(off_flat, cbm2d[:, 0], w1t, w2r, b1r, xs)
    return _undispatch(os_full, pos)


# speedup vs baseline: 4.6907x; 1.2306x over previous
import functools

import jax
import jax.numpy as jnp
from jax.experimental import pallas as pl
from jax.experimental.pallas import tpu as pltpu
from jax.experimental.pallas import tpu_sc as plsc

DEPTH = 11
N_NODES = 2 ** DEPTH - 1
N_LEAVES = 2 ** DEPTH
D = 1024
H = 16
OUT = 1024
BATCH = 8192
BLK = 1024
RGRP = 64
LPB = 16
NLB = N_LEAVES // LPB
CH = 64
OFF_PAD = N_LEAVES + 128
XPAD = BATCH + NLB * CH + CH
NCHUNKS = 256

SC_NC = 2
SC_NS = 16
SC_NW = SC_NC * SC_NS
TPW = BATCH // SC_NW
SC_CH = 32
SC_NCH = TPW // SC_CH


def _routing_body(x_ref, nw_ref, nb_ref, leaves_ref, rank_ref, off_ref, cbm_ref, cnt_ref):
    b = pl.program_id(0)

    @pl.when(b == 0)
    def _():
        cnt_ref[...] = jnp.zeros_like(cnt_ref)

    x = x_ref[...]
    scores = jax.lax.dot_general(
        x, nw_ref[...], (((1,), (1,)), ((), ())),
        precision=jax.lax.Precision.HIGHEST,
        preferred_element_type=jnp.float32)
    scores = scores + nb_ref[...]
    cur = jnp.zeros((BLK, 1), jnp.int32)
    for i in range(DEPTH):
        platform = 2 ** i - 1
        w = 2 ** i
        sl = jax.lax.slice(scores, (0, platform), (BLK, platform + w))
        if i == 0:
            s = sl
        else:
            colw = jax.lax.broadcasted_iota(jnp.int32, (BLK, w), 1)
            onehot = (colw == cur).astype(jnp.float32)
            s = jnp.sum(sl * onehot, axis=1, keepdims=True)
        choice = (s >= 0).astype(jnp.int32)
        cur = cur * 2 + choice
    leaf = cur
    leaves_ref[...] = leaf

    cnt = cnt_ref[...]
    colg = jax.lax.broadcasted_iota(jnp.int32, (RGRP, N_LEAVES), 1)
    rr = jax.lax.broadcasted_iota(jnp.int32, (RGRP, RGRP), 0)
    rc = jax.lax.broadcasted_iota(jnp.int32, (RGRP, RGRP), 1)
    ranks = []
    for g in range(BLK // RGRP):
        lv = jax.lax.slice(leaf, (g * RGRP, 0), ((g + 1) * RGRP, 1))
        m = (colg == lv).astype(jnp.float32)
        prior = jnp.sum(m * cnt, axis=1, keepdims=True)
        eq = (lv == jnp.reshape(lv, (1, RGRP))) & (rr > rc)
        intra = jnp.sum(eq.astype(jnp.float32), axis=1, keepdims=True)
        ranks.append(prior + intra)
        cnt = cnt + jnp.sum(m, axis=0, keepdims=True)
    cnt_ref[...] = cnt
    rank_ref[...] = jnp.concatenate(ranks, axis=0).astype(jnp.int32)

    @pl.when(b == BATCH // BLK - 1)
    def _():
        c = cnt_ref[...]
        lane = jax.lax.broadcasted_iota(jnp.int32, (1, N_LEAVES), 1)
        lmod = lane % LPB

        def shift_from_left(v, k):
            return jnp.concatenate(
                [jnp.zeros((1, k), jnp.float32), jax.lax.slice(v, (0, 0), (1, N_LEAVES - k))],
                axis=1)

        def shift_from_right(v, k):
            return jnp.concatenate(
                [jax.lax.slice(v, (0, k), (1, N_LEAVES)), jnp.zeros((1, k), jnp.float32)],
                axis=1)

        s = c
        k = 1
        while k < LPB:
            s = s + jnp.where(lmod >= k, shift_from_left(s, k), 0.0)
            k *= 2
        excl_within = s - c
        t = s
        k = 1
        while k < LPB:
            t = jnp.maximum(t, jnp.where(lmod + k < LPB, shift_from_right(t, k), 0.0))
            k *= 2
        tpad = jnp.ceil(t / float(CH)) * float(CH)
        u = jnp.where(lmod == LPB - 1, tpad, 0.0)
        acc = u
        k = 1
        while k < N_LEAVES:
            acc = acc + shift_from_left(acc, k)
            k *= 2
        excl_u = acc - u
        off = excl_u + excl_within
        total = jnp.sum(u)
        pad = jnp.full((1, OFF_PAD - N_LEAVES), 1.0, jnp.float32) * total
        off_ref[...] = jnp.concatenate([off, pad], axis=1).astype(jnp.int32)
        kvec = jax.lax.broadcasted_iota(jnp.int32, (NCHUNKS, 1), 0).astype(jnp.float32) * float(CH)
        is_start = (lmod == 0)
        le = ((excl_u <= kvec) & is_start).astype(jnp.float32)
        cbm_ref[...] = (jnp.sum(le, axis=1, keepdims=True) - 1.0).astype(jnp.int32)


def _route(x, node_weights, node_biases):
    nw_pad = jnp.concatenate(
        [node_weights, jnp.zeros((1, D), node_weights.dtype)], axis=0)
    nb_pad = jnp.concatenate(
        [node_biases[:, 0], jnp.zeros((1,), node_biases.dtype)]).reshape(1, N_LEAVES)
    leaves, rank, off, cbm = pl.pallas_call(
        _routing_body,
        grid=(BATCH // BLK,),
        in_specs=[
            pl.BlockSpec((BLK, D), lambda b: (b, 0)),
            pl.BlockSpec((N_LEAVES, D), lambda b: (0, 0)),
            pl.BlockSpec((1, N_LEAVES), lambda b: (0, 0)),
        ],
        out_specs=[
            pl.BlockSpec((BLK, 1), lambda b: (b, 0)),
            pl.BlockSpec((BLK, 1), lambda b: (b, 0)),
            pl.BlockSpec((1, OFF_PAD), lambda b: (0, 0)),
            pl.BlockSpec((NCHUNKS, 1), lambda b: (0, 0)),
        ],
        out_shape=[
            jax.ShapeDtypeStruct((BATCH, 1), jnp.int32),
            jax.ShapeDtypeStruct((BATCH, 1), jnp.int32),
            jax.ShapeDtypeStruct((1, OFF_PAD), jnp.int32),
            jax.ShapeDtypeStruct((NCHUNKS, 1), jnp.int32),
        ],
        scratch_shapes=[pltpu.VMEM((1, N_LEAVES), jnp.float32)],
    )(x, nw_pad, nb_pad)
    return leaves, rank, off, cbm


def _dispatch(x, leaves, rank, offsets):
    mesh = plsc.VectorSubcoreMesh(core_axis_name="c", subcore_axis_name="s")

    @functools.partial(
        pl.kernel,
        mesh=mesh,
        out_type=[
            jax.ShapeDtypeStruct((XPAD, D), jnp.float32),
            jax.ShapeDtypeStruct((BATCH,), jnp.int32),
        ],
        scratch_types=[
            pltpu.VMEM((N_LEAVES,), jnp.int32),
            pltpu.VMEM((SC_CH, D), jnp.float32),
            pltpu.VMEM((SC_NCH, SC_CH), jnp.int32),
            pltpu.VMEM((SC_CH,), jnp.int32),
            pltpu.VMEM((SC_CH,), jnp.int32),
            pltpu.SemaphoreType.DMA,
        ],
        compiler_params=pltpu.CompilerParams(needs_layout_passes=False),
    )
    def body(x_hbm, lv_hbm, rk_hbm, off_hbm, xs_hbm, pos_hbm,
             off_v, rows_v, pos_v, lv_v, rk_v, sem):
        wid = jax.lax.axis_index("s") * SC_NC + jax.lax.axis_index("c")
        pltpu.sync_copy(off_hbm, off_v)
        for i in range(SC_NCH):
            base = wid * TPW + i * SC_CH
            pltpu.sync_copy(lv_hbm.at[pl.ds(base, SC_CH)], lv_v)
            pltpu.sync_copy(rk_hbm.at[pl.ds(base, SC_CH)], rk_v)
            for k in range(0, SC_CH, 16):
                lvv = lv_v[pl.ds(k, 16)]
                offv = plsc.load_gather(off_v, [lvv])
                pos_v[i, pl.ds(k, 16)] = offv + rk_v[pl.ds(k, 16)]
            pltpu.sync_copy(x_hbm.at[pl.ds(base, SC_CH)], rows_v)
            pltpu.async_copy(rows_v, xs_hbm.at[pos_v.at[i]], sem).wait()
            pltpu.sync_copy(pos_v.at[i], pos_hbm.at[pl.ds(base, SC_CH)])

    return body(x, leaves, rank, offsets)


def _expert_body(off_ref, cbm_ref, w1_ref, w2_ref, b1_ref, xs_ref, os_ref):
    k = pl.program_id(0)
    L = cbm_ref[k]
    base = k * CH
    y = jax.lax.dot_general(
        xs_ref[...], w1_ref[...], (((1,), (0,)), ((), ())),
        preferred_element_type=jnp.float32)
    y = y + b1_ref[0]
    a = jax.nn.gelu(y)
    g = base + jax.lax.broadcasted_iota(jnp.int32, (CH, 1), 0)
    masks = []
    for j in range(LPB):
        mj = (g >= off_ref[LPB * L + j]) & (g < off_ref[LPB * L + j + 1])
        masks.append(jnp.broadcast_to(mj, (CH, H)).astype(jnp.float32))
    mask = jnp.concatenate(masks, axis=1)
    am = a * mask
    os_ref[...] = jax.lax.dot_general(
        am, w2_ref[...], (((1,), (0,)), ((), ())),
        preferred_element_type=jnp.float32)


def _experts(off_flat, cbm_flat, w1t, w2r, b1r, xs):
    grid_spec = pltpu.PrefetchScalarGridSpec(
        num_scalar_prefetch=2,
        grid=(NCHUNKS,),
        in_specs=[
            pl.BlockSpec((D, LPB * H), lambda k, off, cbm: (0, cbm[k])),
            pl.BlockSpec((LPB * H, OUT), lambda k, off, cbm: (cbm[k], 0)),
            pl.BlockSpec((1, 1, LPB * H), lambda k, off, cbm: (cbm[k], 0, 0)),
            pl.BlockSpec((CH, D), lambda k, off, cbm: (k, 0)),
        ],
        out_specs=pl.BlockSpec((CH, OUT), lambda k, off, cbm: (k, 0)),
    )
    return pl.pallas_call(
        _expert_body,
        grid_spec=grid_spec,
        out_shape=jax.ShapeDtypeStruct((XPAD, OUT), jnp.float32),
    )(off_flat, cbm_flat, w1t, w2r, b1r, xs)


def _undispatch(os_full, pos):
    mesh = plsc.VectorSubcoreMesh(core_axis_name="c", subcore_axis_name="s")

    @functools.partial(
        pl.kernel,
        mesh=mesh,
        out_type=jax.ShapeDtypeStruct((BATCH, OUT), jnp.float32),
        scratch_types=[
            pltpu.VMEM((SC_CH, OUT), jnp.float32),
            pltpu.VMEM((SC_CH,), jnp.int32),
            pltpu.SemaphoreType.DMA,
        ],
    )
    def body(os_hbm, pos_hbm, out_hbm, rows_v, pos_v, sem):
        wid = jax.lax.axis_index("s") * SC_NC + jax.lax.axis_index("c")
        for i in range(SC_NCH):
            base = wid * TPW + i * SC_CH
            pltpu.sync_copy(pos_hbm.at[pl.ds(base, SC_CH)], pos_v)
            pltpu.async_copy(os_hbm.at[pos_v], rows_v, sem).wait()
            pltpu.sync_copy(rows_v, out_hbm.at[pl.ds(base, SC_CH)])

    return body(os_full, pos)


def kernel(x, node_weights, node_biases, w1s, b1s, w2s):
    leaves2d, rank2d, off2d, cbm2d = _route(x, node_weights, node_biases)
    leaves = leaves2d[:, 0]
    rank = rank2d[:, 0]
    off_flat = off2d[0]
    xs, pos = _dispatch(x, leaves, rank, off_flat[:N_LEAVES])
    w1t = w1s.transpose(1, 0, 2).reshape(D, N_LEAVES * H)
    w2r = w2s.reshape(N_LEAVES * H, OUT)
    b1r = b1s.reshape(NLB, 1, LPB * H)
    os_full = _experts(off_flat, cbm2d[:, 0], w1t, w2r, b1r, xs)
    return _undispatch(os_full, pos)
